# Initial kernel scaffold; baseline (speedup 1.0000x reference)
import jax
import jax.numpy as jnp
from jax.experimental import pallas as pl


def kernel(x, table):
    raise NotImplementedError("write your pallas kernel here")


# speedup vs baseline: 7.1985x; 7.1985x over previous
import functools

import jax
import jax.numpy as jnp
from jax import lax
from jax.experimental import pallas as pl
from jax.experimental.pallas import tpu as pltpu
from jax.experimental.pallas import tpu_sc as plsc


def _make_sc_kernel(S, VP, D):
    info = plsc.get_sparse_core_info()
    NC, NS = info.num_cores, info.num_subcores
    NW = NC * NS
    rows_per_sub = VP // NS
    chunk = 128
    n_chunks = rows_per_sub // chunk
    rows_out = S // NW
    mesh = plsc.VectorSubcoreMesh(core_axis_name="c", subcore_axis_name="s")

    @functools.partial(
        pl.kernel,
        out_type=jax.ShapeDtypeStruct((S, S, D), jnp.float32),
        mesh=mesh,
        scratch_types=[
            pltpu.VMEM((chunk + 8, D), jnp.float32),
            pltpu.VMEM_SHARED((VP + 1408, D), jnp.float32),
            pltpu.SemaphoreType.DMA,
            pltpu.SemaphoreType.DMA,
        ],
    )
    def sc_kernel(tp_hbm, out_hbm, tmp, rev, sem_g, sem_o):
        c = lax.axis_index("c")
        s = lax.axis_index("s")

        for ch in range(n_chunks):
            base = s * rows_per_sub + ch * chunk
            src_lo = (VP - chunk) - base
            pltpu.sync_copy(tp_hbm.at[pl.ds(src_lo, chunk + 8)], tmp)

            def enq(q, carry):
                pltpu.async_copy(
                    tmp.at[pl.ds(q, 1)],
                    rev.at[pl.ds(1408 + base + (chunk - 1) - q, 1)],
                    sem_g,
                )
                return carry

            lax.fori_loop(0, chunk, enq, 0)

            def drain(q, carry):
                pltpu.make_async_copy(tmp.at[0], rev.at[1408 + base], sem_g).wait()
                return carry

            lax.fori_loop(0, chunk, drain, 0)
        plsc.subcore_barrier()

        wid = s * NC + c
        handles = []
        for r in range(rows_out):
            i = wid * rows_out + r
            handles.append(
                pltpu.async_copy(rev.at[pl.ds(1408 + S - i, S)], out_hbm.at[i], sem_o)
            )
            if len(handles) > 8:
                handles.pop(0).wait()
        for h in handles:
            h.wait()

    return sc_kernel


def kernel(x, table):
    S = x.shape[1]
    V, D = table.shape
    tp = jnp.concatenate([table, jnp.zeros((2 * S - V + 8, D), table.dtype)])
    return _make_sc_kernel(S, 2 * S, D)(tp)


# speedup vs baseline: 7.9370x; 1.1026x over previous
import functools

import jax
import jax.numpy as jnp
from jax import lax
from jax.experimental import pallas as pl
from jax.experimental.pallas import tpu as pltpu
from jax.experimental.pallas import tpu_sc as plsc

_SC_SHIFT = 1408
_R_TC = 1024
_TC_BLK = 4


def _make_sc_kernel(S, VP, D, r_tc):
    info = plsc.get_sparse_core_info()
    NC, NS = info.num_cores, info.num_subcores
    NW = NC * NS
    rows_per_sub = VP // NS
    chunk = 128
    n_chunks = rows_per_sub // chunk
    rows_out = (S - r_tc) // NW
    mesh = plsc.VectorSubcoreMesh(core_axis_name="c", subcore_axis_name="s")

    @functools.partial(
        pl.kernel,
        out_type=jax.ShapeDtypeStruct((S, S, D), jnp.float32),
        mesh=mesh,
        scratch_types=[
            pltpu.VMEM((chunk + 8, D), jnp.float32),
            pltpu.VMEM_SHARED((VP + _SC_SHIFT, D), jnp.float32),
            pltpu.SemaphoreType.DMA,
            pltpu.SemaphoreType.DMA,
        ],
    )
    def sc_kernel(tp_hbm, out_hbm, tmp, rev, sem_g, sem_o):
        c = lax.axis_index("c")
        s = lax.axis_index("s")

        for ch in range(n_chunks):
            base = s * rows_per_sub + ch * chunk
            src_lo = (VP - chunk) - base
            pltpu.sync_copy(tp_hbm.at[pl.ds(src_lo, chunk + 8)], tmp)

            def enq(q, carry):
                pltpu.async_copy(
                    tmp.at[pl.ds(q, 1)],
                    rev.at[pl.ds(_SC_SHIFT + base + (chunk - 1) - q, 1)],
                    sem_g,
                )
                return carry

            lax.fori_loop(0, chunk, enq, 0)

            def drain(q, carry):
                pltpu.make_async_copy(tmp.at[0], rev.at[_SC_SHIFT + base], sem_g).wait()
                return carry

            lax.fori_loop(0, chunk, drain, 0)
        plsc.subcore_barrier()

        wid = s * NC + c
        handles = []
        for r in range(rows_out):
            i = r_tc + wid * rows_out + r
            handles.append(
                pltpu.async_copy(
                    rev.at[pl.ds(_SC_SHIFT + S - i, S)], out_hbm.at[i], sem_o
                )
            )
            if len(handles) > 8:
                handles.pop(0).wait()
        for h in handles:
            h.wait()

    return sc_kernel


def _tc_body(rev8_ref, alias_ref, out_ref):
    del alias_ref
    g = pl.program_id(0)
    for r in range(_TC_BLK):
        i = g * _TC_BLK + r
        p = lax.rem(i, 8)
        off = pl.multiple_of((2048 - i) + p, 8)
        out_ref[r] = rev8_ref[p, pl.ds(off, 2048), :]


def _tc_fill(rev8, sc_out, S, D):
    return pl.pallas_call(
        _tc_body,
        grid=(_R_TC // _TC_BLK,),
        in_specs=[
            pl.BlockSpec((8, 2 * S + 8, D), lambda g: (0, 0, 0)),
            pl.BlockSpec(memory_space=pl.ANY),
        ],
        out_specs=pl.BlockSpec((_TC_BLK, S, D), lambda g: (g, 0, 0)),
        out_shape=jax.ShapeDtypeStruct((S, S, D), jnp.float32),
        input_output_aliases={1: 0},
    )(rev8, sc_out)


def kernel(x, table):
    S = x.shape[1]
    V, D = table.shape
    tp = jnp.concatenate([table, jnp.zeros((2 * S - V + 8, D), table.dtype)])

    sc_out = _make_sc_kernel(S, 2 * S, D, _R_TC)(tp)

    rev = tp[: 2 * S][::-1]
    rev8 = jnp.stack(
        [jnp.pad(rev, ((p, 8 - p), (0, 0))) for p in range(8)]
    )
    return _tc_fill(rev8, sc_out, S, D)
